# baseline (speedup 1.0000x reference)
import functools
import jax
import jax.numpy as jnp
from jax import lax
from jax.experimental import pallas as pl
from jax.experimental.pallas import tpu as pltpu
from jax.experimental.pallas import tpu_sc as plsc

_DEG = 3
_OUT = 256
_NCP = 32
_KL = 36
_KPAD = 48
_L = 16
_NW = 32
_ROWS_PW = _OUT // _NW
_ROW_F = _OUT * 3

_f32 = jnp.float32
_i32 = jnp.int32


def _iota16():
    return lax.broadcasted_iota(_i32, (_L,), 0)


def _splat_at(kv_ref, j):
    return plsc.load_gather(kv_ref, [jnp.full((_L,), j, _i32)])


def _normalize(kv_ref):
    iota = _iota16()
    for t in range(3):
        raw = kv_ref[pl.ds(16 * t, 16)]
        kv_ref[pl.ds(16 * t, 16)] = jnp.where(raw < 0.0, _f32(0.0001), raw)
    for t in range(3):
        for s in (1, 2, 4, 8):
            v = kv_ref[pl.ds(16 * t, 16)]
            g = plsc.load_gather(kv_ref, [16 * t + jnp.maximum(iota - s, 0)])
            kv_ref[pl.ds(16 * t, 16)] = v + jnp.where(iota >= s, g, _f32(0.0))
    t0 = _splat_at(kv_ref, 15)
    kv_ref[pl.ds(16, 16)] = kv_ref[pl.ds(16, 16)] + t0
    t1 = _splat_at(kv_ref, 31)
    kv_ref[pl.ds(32, 16)] = kv_ref[pl.ds(32, 16)] + t1
    plsc.store_scatter(kv_ref, [jnp.full((_L,), 44, _i32)],
                       kv_ref[pl.ds(0, 16)], mask=iota == 0)
    k0 = _splat_at(kv_ref, 44)
    klast = _splat_at(kv_ref, _KL - 1)
    denom = klast - k0
    for t in range(3):
        kv_ref[pl.ds(16 * t, 16)] = (kv_ref[pl.ds(16 * t, 16)] - k0) / denom


def _count_prefix(kv_ref, pred):
    pos = jnp.zeros((_L,), _i32)
    for sz in (16, 8, 4, 2, 1):
        npos = pos + sz
        ok = npos <= 30
        idx = jnp.minimum(_DEG + npos - 1, _KL - 1)
        vals = plsc.load_gather(kv_ref, [idx])
        take = ok & pred(vals)
        pos = jnp.where(take, npos, pos)
    return pos


def _span_basis(kv_ref, c):
    iota = _iota16()
    step = _f32((1.0 - 2e-05) / (_OUT - 1))
    ep = (iota + 16 * c).astype(_f32) * step + _f32(1e-05)
    m_cnt = _count_prefix(kv_ref, lambda v: (ep - v) > 1e-08)
    m = jnp.maximum(m_cnt - 1, 0)
    kvm = plsc.load_gather(kv_ref, [m + _DEG])
    fcnt = _count_prefix(kv_ref, lambda v: v < kvm)
    span = jnp.where(m_cnt > 0, _DEG + fcnt, _DEG)
    kvo = {o: plsc.load_gather(kv_ref, [span + o]) for o in range(-2, 4)}
    basis = [jnp.zeros((_L,), _f32) for _ in range(_DEG + 1)]
    basis[0] = jnp.ones((_L,), _f32)
    for k in range(1, _DEG + 1):
        saved = jnp.zeros((_L,), _f32)
        for r in range(k):
            left = kvo[r + 1]
            right = kvo[1 - k + r]
            den = (left - ep) + (ep - right)
            temp = basis[r] / den
            temp = jnp.where(den == 0.0, _f32(0.0001), temp)
            basis[r] = saved + (left - ep) * temp
            saved = (ep - right) * temp
        basis[k] = saved
    return span, basis


def _sc_body(cp_hbm, kvx_hbm, kvy_hbm, out_hbm,
             cp_v, kvx_v, kvy_v, gy_v, wt_v, g_v, out_v):
    w = lax.axis_index("s") * 2 + lax.axis_index("c")
    pltpu.sync_copy(cp_hbm, cp_v)
    pltpu.sync_copy(kvx_hbm, kvx_v)
    pltpu.sync_copy(kvy_hbm, kvy_v)
    _normalize(kvx_v)
    _normalize(kvy_v)
    iota = _iota16()

    def y_chunk(c, _):
        span, b = _span_basis(kvy_v, c)
        for r in range(_DEG + 1):
            yj = lax.rem(span - _DEG - r + _NCP, _NCP)
            gy_v[pl.ds(r * _OUT + 16 * c, 16)] = yj * 48
            wt_v[pl.ds(r * _OUT + 16 * c, 16)] = b[r]
        return 0
    lax.fori_loop(0, 16, y_chunk, 0, unroll=False)

    cx = w >> 1
    spanx, bx = _span_basis(kvx_v, cx)

    base = [lax.rem(spanx - _DEG - l + _NCP, _NCP) * 96 for l in range(_DEG + 1)]

    def g_loop(m, _):
        acc = jnp.zeros((_L,), _f32)
        for l in range(_DEG + 1):
            acc = acc + bx[l] * plsc.load_gather(cp_v, [base[l] + m])
        g_v[pl.ds(16 * m, 16)] = acc
        return 0
    lax.fori_loop(0, 96, g_loop, 0, unroll=False)

    i_loc = (w & 1) * 8
    out3i = iota * 3

    def row_loop(rl, _):
        irow = i_loc + rl

        def chunk_loop(jc, _2):
            accs = [jnp.zeros((_L,), _f32) for _ in range(3)]
            for r in range(_DEG + 1):
                gidx = gy_v[pl.ds(r * _OUT + 16 * jc, 16)]
                wt = wt_v[pl.ds(r * _OUT + 16 * jc, 16)]
                for d in range(3):
                    g = plsc.load_gather(g_v, [gidx + (16 * d + irow)])
                    accs[d] = accs[d] + wt * g
            pos = rl * _ROW_F + 48 * jc + out3i
            for d in range(3):
                plsc.store_scatter(out_v, [pos + d], accs[d])
            return 0
        lax.fori_loop(0, 16, chunk_loop, 0, unroll=False)
        return 0
    lax.fori_loop(0, _ROWS_PW, row_loop, 0, unroll=False)

    pltpu.sync_copy(
        out_v, out_hbm.at[pl.ds(w * (_ROWS_PW * _ROW_F), _ROWS_PW * _ROW_F)])


@jax.jit
def _sc_call(cp, kvx, kvy):
    mesh = plsc.VectorSubcoreMesh(
        core_axis_name="c", subcore_axis_name="s",
        num_cores=2, num_subcores=16)
    return functools.partial(
        pl.kernel,
        mesh=mesh,
        out_type=jax.ShapeDtypeStruct((_OUT * _OUT * 3,), _f32),
        compiler_params=pltpu.CompilerParams(needs_layout_passes=False),
        scratch_types=[
            pltpu.VMEM((_NCP * _NCP * 3,), _f32),
            pltpu.VMEM((_KPAD,), _f32),
            pltpu.VMEM((_KPAD,), _f32),
            pltpu.VMEM((4 * _OUT,), _i32),
            pltpu.VMEM((4 * _OUT,), _f32),
            pltpu.VMEM((96 * _L,), _f32),
            pltpu.VMEM((_ROWS_PW * _ROW_F,), _f32),
        ],
    )(_sc_body)(cp, kvx, kvy)


def kernel(control_points, knot_vector_x, knot_vector_y):
    cp = jnp.reshape(control_points, (-1,))
    kvx = jnp.pad(knot_vector_x[0], (0, _KPAD - _KL))
    kvy = jnp.pad(knot_vector_y[0], (0, _KPAD - _KL))
    out = _sc_call(cp, kvx, kvy)
    return jnp.reshape(out, (1, _OUT, _OUT, 3))


# speedup vs baseline: 13.6411x; 13.6411x over previous
import jax
import jax.numpy as jnp
from jax import lax
from jax.experimental import pallas as pl
from jax.experimental.pallas import tpu as pltpu

_DEG = 3
_OUT = 256
_NCP = 32
_KL = 36
_KP = 128


def _axis_matrix(kv_ref, n_out):
    f32 = jnp.float32
    i32 = jnp.int32

    iota_k = lax.broadcasted_iota(i32, (1, _KP), 1)
    valid = iota_k < _KL
    kv_raw = kv_ref[...]
    kcl = jnp.where(kv_raw < 0.0, 0.0001, kv_raw)
    kcl = jnp.where(valid, kcl, 0.0)

    tri = (
        lax.broadcasted_iota(i32, (_KP, _KP), 0)
        <= lax.broadcasted_iota(i32, (_KP, _KP), 1)
    ).astype(f32)
    kc = jnp.dot(kcl, tri, preferred_element_type=f32, precision=lax.Precision.HIGHEST)

    k0 = kc[:, 0:1]
    klast = kc[:, _KL - 1 : _KL]
    kvn = (kc - k0) / (klast - k0)

    step = (1.0 - 2e-05) / (n_out - 1)
    ep = (
        lax.broadcasted_iota(i32, (n_out, 1), 0).astype(f32) * step + 1e-05
    )

    iota2 = lax.broadcasted_iota(i32, (n_out, _KP), 1)
    diff = ep - kvn
    in_band = (iota2 >= _DEG) & (iota2 < _KL - 2 * _DEG + _DEG)
    masked = jnp.where(diff > 1e-08, diff, 1.0)
    masked = jnp.where(in_band, masked, 2.0)
    minv = jnp.min(masked, axis=1, keepdims=True)
    cand = jnp.where(masked == minv, iota2, _KP + 1)
    span = jnp.min(cand, axis=1, keepdims=True)

    def kv_at(offset):
        oh = (iota2 == span + offset).astype(f32)
        return jnp.sum(oh * kvn, axis=1, keepdims=True)

    kv_off = {o: kv_at(o) for o in range(-2, 4)}

    basis = [jnp.zeros((n_out, 1), f32) for _ in range(_DEG + 1)]
    basis[0] = jnp.ones((n_out, 1), f32)
    for k in range(1, _DEG + 1):
        saved = jnp.zeros((n_out, 1), f32)
        for r in range(k):
            left = kv_off[r + 1]
            right = kv_off[1 - k + r]
            denom = (left - ep) + (ep - right)
            temp = basis[r] / denom
            temp = jnp.where(denom == 0.0, 0.0001, temp)
            basis[r] = saved + (left - ep) * temp
            saved = (ep - right) * temp
        basis[k] = saved

    iota_c = lax.broadcasted_iota(i32, (n_out, _NCP), 1)
    amat = jnp.zeros((n_out, _NCP), f32)
    for l in range(_DEG + 1):
        tgt = lax.rem(span - _DEG - l + _NCP, _NCP)
        amat = amat + jnp.where(iota_c == tgt, basis[l], 0.0)
    return amat


def _body(cp_ref, kvx_ref, kvy_ref, out_ref):
    ax = _axis_matrix(kvx_ref, _OUT)
    ay = _axis_matrix(kvy_ref, _OUT)
    for d in range(3):
        tmp = jnp.dot(ax, cp_ref[d], preferred_element_type=jnp.float32, precision=lax.Precision.HIGHEST)
        out_ref[d] = lax.dot_general(
            tmp, ay, (((1,), (1,)), ((), ())),
            preferred_element_type=jnp.float32,
            precision=lax.Precision.HIGHEST,
        )


def kernel(control_points, knot_vector_x, knot_vector_y):
    cp = jnp.transpose(control_points, (2, 0, 1))
    kvx = jnp.pad(knot_vector_x, ((0, 0), (0, _KP - _KL)))
    kvy = jnp.pad(knot_vector_y, ((0, 0), (0, _KP - _KL)))
    out = pl.pallas_call(
        _body,
        out_shape=jax.ShapeDtypeStruct((3, _OUT, _OUT), jnp.float32),
    )(cp, kvx, kvy)
    return jnp.transpose(out, (1, 2, 0))[None]
